# Initial kernel scaffold; baseline (speedup 1.0000x reference)
import functools

import jax
import jax.numpy as jnp
from jax import lax
from jax.experimental import pallas as pl
from jax.experimental.pallas import tpu as pltpu
from jax.experimental.pallas import tpu_sc as plsc

B = 16384
F = 26
D = 32
OUT_D = 33
NW = 32
RPW = B // NW
CB = 32
NCHUNK = RPW // CB
CR = CB * F

_mesh = plsc.VectorSubcoreMesh(core_axis_name="c", subcore_axis_name="s")


@functools.partial(
    pl.kernel,
    out_type=jax.ShapeDtypeStruct((B, OUT_D), jnp.float32),
    mesh=_mesh,
    scratch_types=[
        pltpu.VMEM((RPW * F,), jnp.int32),
        pltpu.VMEM((CR, D), jnp.float32),
        pltpu.VMEM((CB, OUT_D), jnp.float32),
        pltpu.VMEM((1, D), jnp.float32),
        pltpu.SemaphoreType.DMA,
    ],
)
def _tower(ids_hbm, table_hbm, w_hbm, out_hbm, idx_v, rows_v, out_v, w_v, sem):
    wid = lax.axis_index("s") * 2 + lax.axis_index("c")
    base = wid * (RPW * F)
    pltpu.sync_copy(ids_hbm.at[pl.ds(base, RPW * F)], idx_v)
    pltpu.sync_copy(w_hbm, w_v)
    w0 = w_v[0, pl.ds(0, 16)]
    w1 = w_v[0, pl.ds(16, 16)]

    def chunk_body(g, carry):
        pltpu.async_copy(table_hbm.at[idx_v.at[pl.ds(g * CR, CR)]], rows_v, sem).wait()

        def row_body(b, carry2):
            r0 = b * F
            acc0 = jnp.zeros((16,), jnp.float32)
            acc1 = jnp.zeros((16,), jnp.float32)
            q = jnp.zeros((16,), jnp.float32)
            for f in range(F):
                e0 = rows_v[r0 + f, pl.ds(0, 16)]
                e1 = rows_v[r0 + f, pl.ds(16, 16)]
                acc0 = acc0 + e0
                acc1 = acc1 + e1
                q = q + e0 * e0 + e1 * e1
            t = acc0 * w0 + acc1 * w1 + 0.5 * (acc0 * acc0 + acc1 * acc1) - 0.5 * q
            out_v[b, 0] = jnp.sum(t)
            out_v[b, pl.ds(1, 16)] = acc0
            out_v[b, pl.ds(17, 16)] = acc1
            return carry2

        lax.fori_loop(0, CB, row_body, 0)
        row0 = wid * RPW + g * CB
        pltpu.sync_copy(out_v, out_hbm.at[pl.ds(row0, CB)])
        return carry

    lax.fori_loop(0, NCHUNK, chunk_body, 0)


def kernel(item_feature_ids, emb_table, linear_w):
    ids = item_feature_ids.astype(jnp.int32).reshape(-1)
    return _tower(ids, emb_table, linear_w)


# speedup vs baseline: 1.9381x; 1.9381x over previous
import functools

import jax
import jax.numpy as jnp
from jax import lax
from jax.experimental import pallas as pl
from jax.experimental.pallas import tpu as pltpu
from jax.experimental.pallas import tpu_sc as plsc

B = 16384
F = 26
D = 32
OUT_D = 33
NW = 32
RPW = B // NW
CB = 32
NCHUNK = RPW // CB
CR = CB * F

_mesh = plsc.VectorSubcoreMesh(core_axis_name="c", subcore_axis_name="s")

_GATHER_DNUMS = lax.GatherDimensionNumbers(
    offset_dims=(), collapsed_slice_dims=(0,), start_index_map=(0,)
)


def _permute(v, idx):
    return lax.gather(
        v, idx[:, None], _GATHER_DNUMS, slice_sizes=(1,),
        mode=lax.GatherScatterMode.PROMISE_IN_BOUNDS,
    )


@functools.partial(
    pl.kernel,
    out_type=jax.ShapeDtypeStruct((B * OUT_D,), jnp.float32),
    mesh=_mesh,
    scratch_types=[
        pltpu.VMEM((RPW * F,), jnp.int32),
        pltpu.VMEM((CR, D), jnp.float32),
        pltpu.VMEM((CB * OUT_D,), jnp.float32),
        pltpu.VMEM((1, D), jnp.float32),
        pltpu.SemaphoreType.DMA,
    ],
    compiler_params=pltpu.CompilerParams(use_tc_tiling_on_sc=False),
)
def _tower(ids_hbm, table_hbm, w_hbm, out_hbm, idx_v, rows_v, out_v, w_v, sem):
    wid = lax.axis_index("s") * 2 + lax.axis_index("c")
    base = wid * (RPW * F)
    pltpu.sync_copy(ids_hbm.at[pl.ds(base, RPW * F)], idx_v)
    pltpu.sync_copy(w_hbm, w_v)
    w0 = w_v[0, pl.ds(0, 16)]
    w1 = w_v[0, pl.ds(16, 16)]

    def chunk_body(g, carry):
        pltpu.async_copy(table_hbm.at[idx_v.at[pl.ds(g * CR, CR)]], rows_v, sem).wait()

        def row_body(b, carry2):
            r0 = b * F
            acc0 = jnp.zeros((16,), jnp.float32)
            acc1 = jnp.zeros((16,), jnp.float32)
            q = jnp.zeros((16,), jnp.float32)
            for f in range(F):
                e0 = rows_v[r0 + f, pl.ds(0, 16)]
                e1 = rows_v[r0 + f, pl.ds(16, 16)]
                acc0 = acc0 + e0
                acc1 = acc1 + e1
                q = q + e0 * e0 + e1 * e1
            t = acc0 * w0 + acc1 * w1 + 0.5 * (acc0 * acc0 + acc1 * acc1) - 0.5 * q
            lane = lax.iota(jnp.int32, 16)
            for sh in (8, 4, 2, 1):
                t = t + _permute(t, lane ^ sh)
            o = b * OUT_D
            out_v[pl.ds(o, 16)] = t
            out_v[pl.ds(o + 1, 16)] = acc0
            out_v[pl.ds(o + 17, 16)] = acc1
            return carry2

        lax.fori_loop(0, CB, row_body, 0)
        row0 = wid * RPW + g * CB
        pltpu.sync_copy(out_v, out_hbm.at[pl.ds(row0 * OUT_D, CB * OUT_D)])
        return carry

    lax.fori_loop(0, NCHUNK, chunk_body, 0)


def kernel(item_feature_ids, emb_table, linear_w):
    ids = item_feature_ids.astype(jnp.int32).reshape(-1)
    return _tower(ids, emb_table, linear_w).reshape(B, OUT_D)
